# initial kernel scaffold (unmeasured)
import jax
import jax.numpy as jnp
from jax import lax
from jax.experimental import pallas as pl
from jax.experimental.pallas import tpu as pltpu


def kernel(
    x,
):
    def body(*refs):
        pass

    out_shape = jax.ShapeDtypeStruct(..., jnp.float32)
    return pl.pallas_call(body, out_shape=out_shape)(...)


# baseline (device time: 13387 ns/iter reference)
import jax
import jax.numpy as jnp
from jax import lax
from jax.experimental import pallas as pl
from jax.experimental.pallas import tpu as pltpu


def kernel(x):
    m, n = x.shape

    def body(x_ref, out_ref, comm_ref, send_sems, recv_sems):
        my_x = lax.axis_index("x")
        my_y = lax.axis_index("y")
        y_nbr = (my_x, 1 - my_y)
        x_nbr = (1 - my_x, my_y)

        barrier_sem = pltpu.get_barrier_semaphore()
        for nbr in (y_nbr, x_nbr):
            pl.semaphore_signal(
                barrier_sem, inc=1,
                device_id=nbr, device_id_type=pl.DeviceIdType.MESH,
            )
        pl.semaphore_wait(barrier_sem, 2)

        rdma1 = pltpu.make_async_remote_copy(
            src_ref=x_ref,
            dst_ref=comm_ref.at[0],
            send_sem=send_sems.at[0],
            recv_sem=recv_sems.at[0],
            device_id=y_nbr,
            device_id_type=pl.DeviceIdType.MESH,
        )
        rdma1.start()
        rdma1.wait()
        out_ref[:, :] = x_ref[:, :] + comm_ref[0, :, :]

        rdma2 = pltpu.make_async_remote_copy(
            src_ref=out_ref,
            dst_ref=comm_ref.at[1],
            send_sem=send_sems.at[1],
            recv_sem=recv_sems.at[1],
            device_id=x_nbr,
            device_id_type=pl.DeviceIdType.MESH,
        )
        rdma2.start()
        rdma2.wait()
        out_ref[:, :] = out_ref[:, :] + comm_ref[1, :, :]

    return pl.pallas_call(
        body,
        out_shape=jax.ShapeDtypeStruct((m, n), x.dtype),
        in_specs=[pl.BlockSpec(memory_space=pltpu.VMEM)],
        out_specs=pl.BlockSpec(memory_space=pltpu.VMEM),
        scratch_shapes=[
            pltpu.VMEM((2, m, n), x.dtype),
            pltpu.SemaphoreType.DMA((2,)),
            pltpu.SemaphoreType.DMA((2,)),
        ],
        compiler_params=pltpu.CompilerParams(collective_id=0),
    )(x)


# device time: 10616 ns/iter; 1.2610x vs baseline; 1.2610x over previous
import jax
import jax.numpy as jnp
from jax import lax
from jax.experimental import pallas as pl
from jax.experimental.pallas import tpu as pltpu


def kernel(x):
    m, n = x.shape
    hm = m // 2

    def body(x_ref, out_ref, part_ref, comm_ref, send_sems, recv_sems):
        my_x = lax.axis_index("x")
        my_y = lax.axis_index("y")
        y_nbr = (my_x, 1 - my_y)
        x_nbr = (1 - my_x, my_y)

        barrier_sem = pltpu.get_barrier_semaphore()
        for nbr in (y_nbr, x_nbr):
            pl.semaphore_signal(
                barrier_sem, inc=1,
                device_id=nbr, device_id_type=pl.DeviceIdType.MESH,
            )
        pl.semaphore_wait(barrier_sem, 2)

        def exchange(src, dst_slot, sem_slot, dev):
            return pltpu.make_async_remote_copy(
                src_ref=src,
                dst_ref=comm_ref.at[dst_slot],
                send_sem=send_sems.at[sem_slot],
                recv_sem=recv_sems.at[sem_slot],
                device_id=dev,
                device_id_type=pl.DeviceIdType.MESH,
            )

        rows_a = pl.ds(0, hm)
        rows_b = pl.ds(hm, hm)

        a1 = exchange(x_ref.at[rows_a], 0, 0, y_nbr)
        a1.start()
        b1 = exchange(x_ref.at[rows_b], 1, 1, x_nbr)
        b1.start()

        a1.wait_recv()
        part_ref[0, :, :] = x_ref[rows_a, :] + comm_ref[0, :, :]
        a2 = exchange(part_ref.at[0], 2, 2, x_nbr)
        a2.start()

        b1.wait_recv()
        part_ref[1, :, :] = x_ref[rows_b, :] + comm_ref[1, :, :]
        b2 = exchange(part_ref.at[1], 3, 3, y_nbr)
        b2.start()

        a2.wait_recv()
        out_ref[rows_a, :] = part_ref[0, :, :] + comm_ref[2, :, :]
        b2.wait_recv()
        out_ref[rows_b, :] = part_ref[1, :, :] + comm_ref[3, :, :]

        a1.wait_send()
        b1.wait_send()
        a2.wait_send()
        b2.wait_send()

    return pl.pallas_call(
        body,
        out_shape=jax.ShapeDtypeStruct((m, n), x.dtype),
        in_specs=[pl.BlockSpec(memory_space=pltpu.VMEM)],
        out_specs=pl.BlockSpec(memory_space=pltpu.VMEM),
        scratch_shapes=[
            pltpu.VMEM((2, hm, n), x.dtype),
            pltpu.VMEM((4, hm, n), x.dtype),
            pltpu.SemaphoreType.DMA((4,)),
            pltpu.SemaphoreType.DMA((4,)),
        ],
        compiler_params=pltpu.CompilerParams(collective_id=0),
    )(x)


# device time: 9743 ns/iter; 1.3740x vs baseline; 1.0896x over previous
import jax
import jax.numpy as jnp
from jax import lax
from jax.experimental import pallas as pl
from jax.experimental.pallas import tpu as pltpu

CHUNKS = 2


def kernel(x):
    m, n = x.shape
    n_chunks = 2 * CHUNKS
    cm = m // n_chunks

    def body(x_ref, out_ref, part_ref, comm_ref, send_sems, recv_sems):
        my_x = lax.axis_index("x")
        my_y = lax.axis_index("y")
        y_nbr = (my_x, 1 - my_y)
        x_nbr = (1 - my_x, my_y)

        barrier_sem = pltpu.get_barrier_semaphore()
        for nbr in (y_nbr, x_nbr):
            pl.semaphore_signal(
                barrier_sem, inc=1,
                device_id=nbr, device_id_type=pl.DeviceIdType.MESH,
            )
        pl.semaphore_wait(barrier_sem, 2)

        def exchange(src, slot, dev):
            return pltpu.make_async_remote_copy(
                src_ref=src,
                dst_ref=comm_ref.at[slot],
                send_sem=send_sems.at[slot],
                recv_sem=recv_sems.at[slot],
                device_id=dev,
                device_id_type=pl.DeviceIdType.MESH,
            )

        def dims(c):
            return (y_nbr, x_nbr) if c % 2 == 0 else (x_nbr, y_nbr)

        p1 = []
        for c in range(n_chunks):
            r = exchange(x_ref.at[pl.ds(c * cm, cm)], c, dims(c)[0])
            r.start()
            p1.append(r)

        p2 = []
        for c in range(n_chunks):
            p1[c].wait_recv()
            part_ref[c, :, :] = (
                x_ref[pl.ds(c * cm, cm), :] + comm_ref[c, :, :]
            )
            r = exchange(part_ref.at[c], n_chunks + c, dims(c)[1])
            r.start()
            p2.append(r)

        for c in range(n_chunks):
            p2[c].wait_recv()
            out_ref[pl.ds(c * cm, cm), :] = (
                part_ref[c, :, :] + comm_ref[n_chunks + c, :, :]
            )

        for r in p1 + p2:
            r.wait_send()

    return pl.pallas_call(
        body,
        out_shape=jax.ShapeDtypeStruct((m, n), x.dtype),
        in_specs=[pl.BlockSpec(memory_space=pltpu.VMEM)],
        out_specs=pl.BlockSpec(memory_space=pltpu.VMEM),
        scratch_shapes=[
            pltpu.VMEM((n_chunks, cm, n), x.dtype),
            pltpu.VMEM((2 * n_chunks, cm, n), x.dtype),
            pltpu.SemaphoreType.DMA((2 * n_chunks,)),
            pltpu.SemaphoreType.DMA((2 * n_chunks,)),
        ],
        compiler_params=pltpu.CompilerParams(collective_id=0),
    )(x)


# device time: 9507 ns/iter; 1.4081x vs baseline; 1.0248x over previous
import jax
import jax.numpy as jnp
from jax import lax
from jax.experimental import pallas as pl
from jax.experimental.pallas import tpu as pltpu

CHUNKS = 4


def kernel(x):
    m, n = x.shape
    n_chunks = 2 * CHUNKS
    cm = m // n_chunks

    def body(x_ref, out_ref, part_ref, comm_ref, send_sems, recv_sems):
        my_x = lax.axis_index("x")
        my_y = lax.axis_index("y")
        y_nbr = (my_x, 1 - my_y)
        x_nbr = (1 - my_x, my_y)

        barrier_sem = pltpu.get_barrier_semaphore()
        for nbr in (y_nbr, x_nbr):
            pl.semaphore_signal(
                barrier_sem, inc=1,
                device_id=nbr, device_id_type=pl.DeviceIdType.MESH,
            )
        pl.semaphore_wait(barrier_sem, 2)

        def exchange(src, slot, dev):
            return pltpu.make_async_remote_copy(
                src_ref=src,
                dst_ref=comm_ref.at[slot],
                send_sem=send_sems.at[slot],
                recv_sem=recv_sems.at[slot],
                device_id=dev,
                device_id_type=pl.DeviceIdType.MESH,
            )

        def dims(c):
            return (y_nbr, x_nbr) if c % 2 == 0 else (x_nbr, y_nbr)

        p1 = []
        for c in range(n_chunks):
            r = exchange(x_ref.at[pl.ds(c * cm, cm)], c, dims(c)[0])
            r.start()
            p1.append(r)

        p2 = []
        for c in range(n_chunks):
            p1[c].wait_recv()
            part_ref[c, :, :] = (
                x_ref[pl.ds(c * cm, cm), :] + comm_ref[c, :, :]
            )
            r = exchange(part_ref.at[c], n_chunks + c, dims(c)[1])
            r.start()
            p2.append(r)

        for c in range(n_chunks):
            p2[c].wait_recv()
            out_ref[pl.ds(c * cm, cm), :] = (
                part_ref[c, :, :] + comm_ref[n_chunks + c, :, :]
            )

        for r in p1 + p2:
            r.wait_send()

    return pl.pallas_call(
        body,
        out_shape=jax.ShapeDtypeStruct((m, n), x.dtype),
        in_specs=[pl.BlockSpec(memory_space=pltpu.VMEM)],
        out_specs=pl.BlockSpec(memory_space=pltpu.VMEM),
        scratch_shapes=[
            pltpu.VMEM((n_chunks, cm, n), x.dtype),
            pltpu.VMEM((2 * n_chunks, cm, n), x.dtype),
            pltpu.SemaphoreType.DMA((2 * n_chunks,)),
            pltpu.SemaphoreType.DMA((2 * n_chunks,)),
        ],
        compiler_params=pltpu.CompilerParams(collective_id=0),
    )(x)
